# Initial kernel scaffold; baseline (speedup 1.0000x reference)
import functools

import jax
import jax.numpy as jnp
from jax import lax
from jax.experimental import pallas as pl
from jax.experimental.pallas import tpu as pltpu
from jax.experimental.pallas import tpu_sc as plsc

T = 2
N = 10000
E = 320000
D = 128

NC = 2
NS = 16
NW = NC * NS

NP = 10240
CHUNK = 128
EPT = 10112
NCH = EPT // CHUNK
E_PAD = NW * EPT

ZR = 64
ROWS_PER_TILE = NP // NS
DEG_SLICE = T * NP // NS

BLK = 1280
GRID = NP // BLK

_mesh = plsc.VectorSubcoreMesh(
    core_axis_name="c", subcore_axis_name="s", num_cores=NC, num_subcores=NS)


@functools.partial(
    pl.kernel,
    out_type=jax.ShapeDtypeStruct((NC, T * NP), jnp.float32),
    mesh=_mesh,
    scratch_types=[
        pltpu.VMEM((T * NCH, CHUNK), jnp.int32),
        pltpu.VMEM((CHUNK,), jnp.float32),
        pltpu.VMEM((DEG_SLICE,), jnp.float32),
        pltpu.VMEM_SHARED((T * NP,), jnp.float32),
        pltpu.SemaphoreType.DMA,
    ],
)
def _deg_kernel(dst_hbm, out_hbm, idx_v, ones_v, buf_v, deg_sh, sem):
    c = lax.axis_index("c")
    s = lax.axis_index("s")
    wid = s * NC + c
    for i in range(CHUNK // 16):
        ones_v[pl.ds(i * 16, 16)] = jnp.ones((16,), jnp.float32)
    for i in range(DEG_SLICE // 16):
        buf_v[pl.ds(i * 16, 16)] = jnp.zeros((16,), jnp.float32)
    pltpu.sync_copy(buf_v, deg_sh.at[pl.ds(s * DEG_SLICE, DEG_SLICE)])
    pltpu.sync_copy(dst_hbm.at[wid], idx_v)
    plsc.subcore_barrier()

    @pl.loop(0, T * NCH)
    def _scatter(j):
        pltpu.sync_copy(ones_v, deg_sh.at[idx_v.at[j]], add=True)

    plsc.subcore_barrier()
    pltpu.sync_copy(deg_sh.at[pl.ds(s * DEG_SLICE, DEG_SLICE)], buf_v)
    pltpu.sync_copy(buf_v, out_hbm.at[c, pl.ds(s * DEG_SLICE, DEG_SLICE)])


@functools.partial(
    pl.kernel,
    out_type=jax.ShapeDtypeStruct((NC, NP, D), jnp.float32),
    mesh=_mesh,
    scratch_types=[
        pltpu.VMEM((NCH, CHUNK), jnp.int32),
        pltpu.VMEM((NCH, CHUNK), jnp.int32),
        pltpu.VMEM((2, CHUNK, D), jnp.float32),
        pltpu.VMEM((ZR, D), jnp.float32),
        pltpu.VMEM_SHARED((NP, D), jnp.float32),
        pltpu.SemaphoreType.DMA,
    ],
)
def _agg_kernel(hs_hbm, src_hbm, dst_hbm, out_hbm, srcv, dstv, rows, zbuf,
                acc, gsem):
    c = lax.axis_index("c")
    s = lax.axis_index("s")
    wid = s * NC + c
    for r in range(ZR):
        for k in range(D // 16):
            zbuf[r, pl.ds(k * 16, 16)] = jnp.zeros((16,), jnp.float32)
    base = s * ROWS_PER_TILE
    for i in range(ROWS_PER_TILE // ZR):
        pltpu.sync_copy(zbuf, acc.at[pl.ds(base + i * ZR, ZR)])
    pltpu.sync_copy(src_hbm.at[wid], srcv)
    pltpu.sync_copy(dst_hbm.at[wid], dstv)
    plsc.subcore_barrier()

    pltpu.async_copy(hs_hbm.at[srcv.at[0]], rows.at[0], gsem)

    @pl.loop(0, NCH)
    def _edges(j):
        @pl.when(j + 1 < NCH)
        def _():
            pltpu.async_copy(hs_hbm.at[srcv.at[j + 1]], rows.at[(j + 1) % 2],
                             gsem)
        pltpu.make_async_copy(hs_hbm.at[srcv.at[j]], rows.at[j % 2],
                              gsem).wait()
        pltpu.sync_copy(rows.at[j % 2], acc.at[dstv.at[j]], add=True)

    plsc.subcore_barrier()
    for i in range(ROWS_PER_TILE // ZR):
        pltpu.sync_copy(acc.at[pl.ds(base + i * ZR, ZR)], zbuf)
        pltpu.sync_copy(zbuf, out_hbm.at[c].at[pl.ds(base + i * ZR, ZR)])


def _row_spec():
    return pl.BlockSpec((BLK, D), lambda i: (i, 0))


def _col_spec():
    return pl.BlockSpec((BLK, 1), lambda i: (i, 0))


def _full_spec(shape):
    return pl.BlockSpec(shape, lambda i: tuple(0 for _ in shape))


def _tc_scale_mm_body(x_ref, w_ref, p0_ref, p1_ref, o_ref):
    dinv = lax.rsqrt(p0_ref[...] + p1_ref[...] + 1.0)
    h = jnp.dot(x_ref[...], w_ref[...], preferred_element_type=jnp.float32)
    o_ref[...] = h * dinv


def _tc_scale_mm(x, w, p0, p1):
    return pl.pallas_call(
        _tc_scale_mm_body,
        grid=(GRID,),
        in_specs=[_row_spec(), _full_spec((D, D)), _col_spec(), _col_spec()],
        out_specs=_row_spec(),
        out_shape=jax.ShapeDtypeStruct((NP, D), jnp.float32),
    )(x, w, p0, p1)


def _tc_mid_body(q0_ref, q1_ref, hs_ref, p0_ref, p1_ref, b_ref, w_ref, o_ref):
    dinv = lax.rsqrt(p0_ref[...] + p1_ref[...] + 1.0)
    o1 = dinv * (q0_ref[...] + q1_ref[...] + hs_ref[...]) + b_ref[...]
    o1 = jnp.maximum(o1, 0.0)
    h = jnp.dot(o1, w_ref[...], preferred_element_type=jnp.float32)
    o_ref[...] = h * dinv


def _tc_mid(q0, q1, hs, p0, p1, b, w):
    return pl.pallas_call(
        _tc_mid_body,
        grid=(GRID,),
        in_specs=[_row_spec(), _row_spec(), _row_spec(), _col_spec(),
                  _col_spec(), _full_spec((1, D)), _full_spec((D, D))],
        out_specs=_row_spec(),
        out_shape=jax.ShapeDtypeStruct((NP, D), jnp.float32),
    )(q0, q1, hs, p0, p1, b, w)


def _tc_final_body(q0_ref, q1_ref, hs_ref, p0_ref, p1_ref, b_ref, wc_ref,
                   bc_ref, o_ref, imp_ref):
    dinv = lax.rsqrt(p0_ref[...] + p1_ref[...] + 1.0)
    o2 = dinv * (q0_ref[...] + q1_ref[...] + hs_ref[...]) + b_ref[...]
    o_ref[...] = o2
    imp_ref[...] = jnp.dot(o2, wc_ref[...],
                           preferred_element_type=jnp.float32) + bc_ref[...]


def _tc_final(q0, q1, hs, p0, p1, b, wc, bc):
    return pl.pallas_call(
        _tc_final_body,
        grid=(GRID,),
        in_specs=[_row_spec(), _row_spec(), _row_spec(), _col_spec(),
                  _col_spec(), _full_spec((1, D)), _full_spec((D, 1)),
                  _full_spec((1, 1))],
        out_specs=[_row_spec(), _col_spec()],
        out_shape=[jax.ShapeDtypeStruct((NP, D), jnp.float32),
                   jax.ShapeDtypeStruct((NP, 1), jnp.float32)],
    )(q0, q1, hs, p0, p1, b, wc, bc)


def kernel(x_seq, edge_indices, W1, b1, W2, b2, Wc, bc):
    x_pad = jnp.pad(x_seq, ((0, 0), (0, NP - N), (0, 0)))
    pad_idx = N + (jnp.arange(E_PAD - E, dtype=jnp.int32) % (NP - N))

    def _tiled(a):
        return jnp.concatenate([a, pad_idx]).reshape(NW, NCH, CHUNK)

    src_t = [_tiled(edge_indices[t, 0]) for t in range(T)]
    dst_t = [_tiled(edge_indices[t, 1]) for t in range(T)]
    deg_idx = jnp.concatenate([dst_t[0], dst_t[1] + NP], axis=1)

    degp = _deg_kernel(deg_idx)
    b1r = b1.reshape(1, D)
    b2r = b2.reshape(1, D)
    bcr = bc.reshape(1, 1)

    outs = []
    imp = None
    for t in range(T):
        p0 = degp[0, t * NP:(t + 1) * NP, None]
        p1 = degp[1, t * NP:(t + 1) * NP, None]
        hs1 = _tc_scale_mm(x_pad[t], W1, p0, p1)
        q = _agg_kernel(hs1, src_t[t], dst_t[t])
        hs2 = _tc_mid(q[0], q[1], hs1, p0, p1, b1r, W2)
        q2 = _agg_kernel(hs2, src_t[t], dst_t[t])
        o2, imp_t = _tc_final(q2[0], q2[1], hs2, p0, p1, b2r, Wc, bcr)
        outs.append(o2[:N])
        imp = imp_t
    return imp[:N, 0], jnp.stack(outs)


# speedup vs baseline: 24.8951x; 24.8951x over previous
import functools

import jax
import jax.numpy as jnp
from jax import lax
from jax.experimental import pallas as pl
from jax.experimental.pallas import tpu as pltpu
from jax.experimental.pallas import tpu_sc as plsc

T = 2
N = 10000
E = 320000
D = 128

NC = 2
NS = 16
NW = NC * NS

NP = 10240
CHUNK = 128
EPT = 10112
NCH = EPT // CHUNK
E_PAD = NW * EPT

ZR = 8
IDX_BLK = 40
PHASES = ((0, 40), (40, 39))
ROWS_PER_TILE = NP // NS
DEG_SLICE = T * NP // NS

BLK = 1280
GRID = NP // BLK

_mesh = plsc.VectorSubcoreMesh(
    core_axis_name="c", subcore_axis_name="s", num_cores=NC, num_subcores=NS)


@functools.partial(
    pl.kernel,
    out_type=jax.ShapeDtypeStruct((NC, T * NP), jnp.float32),
    mesh=_mesh,
    scratch_types=[
        pltpu.VMEM((T * NCH, CHUNK), jnp.int32),
        pltpu.VMEM((CHUNK,), jnp.float32),
        pltpu.VMEM((DEG_SLICE,), jnp.float32),
        pltpu.VMEM_SHARED((T * NP,), jnp.float32),
        pltpu.SemaphoreType.DMA,
    ],
)
def _deg_kernel(dst_hbm, out_hbm, idx_v, ones_v, buf_v, deg_sh, sem):
    c = lax.axis_index("c")
    s = lax.axis_index("s")
    wid = s * NC + c
    for i in range(CHUNK // 16):
        ones_v[pl.ds(i * 16, 16)] = jnp.ones((16,), jnp.float32)
    for i in range(DEG_SLICE // 16):
        buf_v[pl.ds(i * 16, 16)] = jnp.zeros((16,), jnp.float32)
    pltpu.sync_copy(buf_v, deg_sh.at[pl.ds(s * DEG_SLICE, DEG_SLICE)])
    pltpu.sync_copy(dst_hbm.at[wid], idx_v)
    plsc.subcore_barrier()

    @pl.loop(0, T * NCH)
    def _scatter(j):
        pltpu.sync_copy(ones_v, deg_sh.at[idx_v.at[j]], add=True)

    plsc.subcore_barrier()
    pltpu.sync_copy(deg_sh.at[pl.ds(s * DEG_SLICE, DEG_SLICE)], buf_v)
    pltpu.sync_copy(buf_v, out_hbm.at[c, pl.ds(s * DEG_SLICE, DEG_SLICE)])


@functools.partial(
    pl.kernel,
    out_type=jax.ShapeDtypeStruct((NC, NP, D), jnp.float32),
    mesh=_mesh,
    scratch_types=[
        pltpu.VMEM((IDX_BLK, CHUNK), jnp.int32),
        pltpu.VMEM((IDX_BLK, CHUNK), jnp.int32),
        pltpu.VMEM((2, CHUNK, D), jnp.float32),
        pltpu.VMEM((ZR, D), jnp.float32),
        pltpu.VMEM_SHARED((NP, D), jnp.float32),
        pltpu.SemaphoreType.DMA,
    ],
)
def _agg_kernel(hs_hbm, src_hbm, dst_hbm, out_hbm, srcv, dstv, rows, zbuf,
                acc, gsem):
    c = lax.axis_index("c")
    s = lax.axis_index("s")
    wid = s * NC + c
    for r in range(ZR):
        for k in range(D // 16):
            zbuf[r, pl.ds(k * 16, 16)] = jnp.zeros((16,), jnp.float32)
    base = s * ROWS_PER_TILE
    for i in range(ROWS_PER_TILE // ZR):
        pltpu.sync_copy(zbuf, acc.at[pl.ds(base + i * ZR, ZR)])
    plsc.subcore_barrier()

    for off, cnt in PHASES:
        pltpu.sync_copy(src_hbm.at[wid, pl.ds(off, cnt)],
                        srcv.at[pl.ds(0, cnt)])
        pltpu.sync_copy(dst_hbm.at[wid, pl.ds(off, cnt)],
                        dstv.at[pl.ds(0, cnt)])
        pltpu.async_copy(hs_hbm.at[srcv.at[0]], rows.at[0], gsem)

        @pl.loop(0, cnt)
        def _edges(j):
            @pl.when(j + 1 < cnt)
            def _():
                pltpu.async_copy(hs_hbm.at[srcv.at[j + 1]],
                                 rows.at[(j + 1) % 2], gsem)
            pltpu.make_async_copy(hs_hbm.at[srcv.at[j]], rows.at[j % 2],
                                  gsem).wait()
            pltpu.sync_copy(rows.at[j % 2], acc.at[dstv.at[j]], add=True)

    plsc.subcore_barrier()
    for i in range(ROWS_PER_TILE // ZR):
        pltpu.sync_copy(acc.at[pl.ds(base + i * ZR, ZR)], zbuf)
        pltpu.sync_copy(zbuf, out_hbm.at[c].at[pl.ds(base + i * ZR, ZR)])


def _row_spec():
    return pl.BlockSpec((BLK, D), lambda i: (i, 0))


def _col_spec():
    return pl.BlockSpec((BLK, 1), lambda i: (i, 0))


def _full_spec(shape):
    return pl.BlockSpec(shape, lambda i: tuple(0 for _ in shape))


def _tc_scale_mm_body(x_ref, w_ref, p0_ref, p1_ref, o_ref):
    dinv = lax.rsqrt(p0_ref[...] + p1_ref[...] + 1.0)
    h = jnp.dot(x_ref[...], w_ref[...], preferred_element_type=jnp.float32)
    o_ref[...] = h * dinv


def _tc_scale_mm(x, w, p0, p1):
    return pl.pallas_call(
        _tc_scale_mm_body,
        grid=(GRID,),
        in_specs=[_row_spec(), _full_spec((D, D)), _col_spec(), _col_spec()],
        out_specs=_row_spec(),
        out_shape=jax.ShapeDtypeStruct((NP, D), jnp.float32),
    )(x, w, p0, p1)


def _tc_mid_body(q0_ref, q1_ref, hs_ref, p0_ref, p1_ref, b_ref, w_ref, o_ref):
    dinv = lax.rsqrt(p0_ref[...] + p1_ref[...] + 1.0)
    o1 = dinv * (q0_ref[...] + q1_ref[...] + hs_ref[...]) + b_ref[...]
    o1 = jnp.maximum(o1, 0.0)
    h = jnp.dot(o1, w_ref[...], preferred_element_type=jnp.float32)
    o_ref[...] = h * dinv


def _tc_mid(q0, q1, hs, p0, p1, b, w):
    return pl.pallas_call(
        _tc_mid_body,
        grid=(GRID,),
        in_specs=[_row_spec(), _row_spec(), _row_spec(), _col_spec(),
                  _col_spec(), _full_spec((1, D)), _full_spec((D, D))],
        out_specs=_row_spec(),
        out_shape=jax.ShapeDtypeStruct((NP, D), jnp.float32),
    )(q0, q1, hs, p0, p1, b, w)


def _tc_final_body(q0_ref, q1_ref, hs_ref, p0_ref, p1_ref, b_ref, wc_ref,
                   bc_ref, o_ref, imp_ref):
    dinv = lax.rsqrt(p0_ref[...] + p1_ref[...] + 1.0)
    o2 = dinv * (q0_ref[...] + q1_ref[...] + hs_ref[...]) + b_ref[...]
    o_ref[...] = o2
    imp_ref[...] = jnp.dot(o2, wc_ref[...],
                           preferred_element_type=jnp.float32) + bc_ref[...]


def _tc_final(q0, q1, hs, p0, p1, b, wc, bc):
    return pl.pallas_call(
        _tc_final_body,
        grid=(GRID,),
        in_specs=[_row_spec(), _row_spec(), _row_spec(), _col_spec(),
                  _col_spec(), _full_spec((1, D)), _full_spec((D, 1)),
                  _full_spec((1, 1))],
        out_specs=[_row_spec(), _col_spec()],
        out_shape=[jax.ShapeDtypeStruct((NP, D), jnp.float32),
                   jax.ShapeDtypeStruct((NP, 1), jnp.float32)],
    )(q0, q1, hs, p0, p1, b, wc, bc)


def kernel(x_seq, edge_indices, W1, b1, W2, b2, Wc, bc):
    x_pad = jnp.pad(x_seq, ((0, 0), (0, NP - N), (0, 0)))
    pad_idx = N + (jnp.arange(E_PAD - E, dtype=jnp.int32) % (NP - N))

    def _tiled(a):
        return jnp.concatenate([a, pad_idx]).reshape(NW, NCH, CHUNK)

    src_t = [_tiled(edge_indices[t, 0]) for t in range(T)]
    dst_t = [_tiled(edge_indices[t, 1]) for t in range(T)]
    deg_idx = jnp.concatenate([dst_t[0], dst_t[1] + NP], axis=1)

    degp = _deg_kernel(deg_idx)
    b1r = b1.reshape(1, D)
    b2r = b2.reshape(1, D)
    bcr = bc.reshape(1, 1)

    outs = []
    imp = None
    for t in range(T):
        p0 = degp[0, t * NP:(t + 1) * NP, None]
        p1 = degp[1, t * NP:(t + 1) * NP, None]
        hs1 = _tc_scale_mm(x_pad[t], W1, p0, p1)
        q = _agg_kernel(hs1, src_t[t], dst_t[t])
        hs2 = _tc_mid(q[0], q[1], hs1, p0, p1, b1r, W2)
        q2 = _agg_kernel(hs2, src_t[t], dst_t[t])
        o2, imp_t = _tc_final(q2[0], q2[1], hs2, p0, p1, b2r, Wc, bcr)
        outs.append(o2[:N])
        imp = imp_t
    return imp[:N, 0], jnp.stack(outs)
